# Initial kernel scaffold; baseline (speedup 1.0000x reference)
import jax
import jax.numpy as jnp
from jax.experimental import pallas as pl


def kernel(x, w_gate, W1, b1, W2, b2):
    raise NotImplementedError("write your pallas kernel here")


# speedup vs baseline: 1.8005x; 1.8005x over previous
import functools

import jax
import jax.numpy as jnp
from jax.experimental import pallas as pl
from jax.experimental.pallas import tpu as pltpu

D_MODEL = 1024
N_EXPERTS = 8
TOP_K = 2
FFN = 4096
SEQ = 2048
EPAD = 128
FB = 512
NFB = FFN // FB


def _erf(x):
    a1, a2, a3, a4, a5 = (0.254829592, -0.284496736, 1.421413741,
                          -1.453152027, 1.061405429)
    p = 0.3275911
    ax = jnp.abs(x)
    t = 1.0 / (1.0 + p * ax)
    poly = ((((a5 * t + a4) * t + a3) * t + a2) * t + a1) * t
    y = 1.0 - poly * jnp.exp(-ax * ax)
    return jnp.sign(x) * y


def _gelu(x):
    return 0.5 * x * (1.0 + _erf(x * 0.7071067811865476))


def _gating_body(x_ref, wg_ref, wfull_ref):
    x = x_ref[...]
    logits = jnp.dot(x, wg_ref[...], preferred_element_type=jnp.float32)
    col = jax.lax.broadcasted_iota(jnp.int32, (SEQ, EPAD), 1)
    valid = col < N_EXPERTS
    neg = jnp.float32(-1e30)
    l = jnp.where(valid, logits, neg)
    m1 = jnp.max(l, axis=1, keepdims=True)
    i1 = jnp.min(jnp.where(l == m1, col, EPAD), axis=1, keepdims=True)
    l2 = jnp.where(col == i1, neg, l)
    m2 = jnp.max(l2, axis=1, keepdims=True)
    i2 = jnp.min(jnp.where(l2 == m2, col, EPAD), axis=1, keepdims=True)
    s = jnp.where(valid, jnp.exp(l - m1), 0.0)
    den = jnp.sum(s, axis=1, keepdims=True)
    p1 = 1.0 / den
    p2 = jnp.exp(m2 - m1) / den
    wfull = jnp.where(col == i1, p1, 0.0) + jnp.where(col == i2, p2, 0.0)
    wfull_ref[...] = wfull


def _ffn_body(x_ref, w1_ref, b1_ref, w2_ref, b2_ref, wfull_ref, out_ref,
              acc_ref):
    e = pl.program_id(0)
    f = pl.program_id(1)
    h = jnp.dot(x_ref[...], w1_ref[0], preferred_element_type=jnp.float32)
    h = _gelu(h + b1_ref[0])
    part = jnp.dot(h, w2_ref[0], preferred_element_type=jnp.float32)

    @pl.when(f == 0)
    def _():
        acc_ref[...] = part

    @pl.when(f > 0)
    def _():
        acc_ref[...] += part

    @pl.when(f == NFB - 1)
    def _():
        col = jax.lax.broadcasted_iota(jnp.int32, (SEQ, EPAD), 1)
        w = jnp.sum(jnp.where(col == e, wfull_ref[...], 0.0), axis=1,
                    keepdims=True)
        y = (acc_ref[...] + b2_ref[0]) * w

        @pl.when(e == 0)
        def _():
            out_ref[...] = y

        @pl.when(e > 0)
        def _():
            out_ref[...] += y


def _moe(x2d, wg_p, W1, b1r, W2, b2r, interpret=False):
    wfull = pl.pallas_call(
        _gating_body,
        out_shape=jax.ShapeDtypeStruct((SEQ, EPAD), jnp.float32),
        interpret=interpret,
    )(x2d, wg_p)

    out = pl.pallas_call(
        _ffn_body,
        grid=(N_EXPERTS, NFB),
        in_specs=[
            pl.BlockSpec((SEQ, D_MODEL), lambda e, f: (0, 0)),
            pl.BlockSpec((1, D_MODEL, FB), lambda e, f: (e, 0, f)),
            pl.BlockSpec((1, 1, FB), lambda e, f: (e, 0, f)),
            pl.BlockSpec((1, FB, D_MODEL), lambda e, f: (e, f, 0)),
            pl.BlockSpec((1, 1, D_MODEL), lambda e, f: (e, 0, 0)),
            pl.BlockSpec((SEQ, EPAD), lambda e, f: (0, 0)),
        ],
        out_specs=pl.BlockSpec((SEQ, D_MODEL), lambda e, f: (0, 0)),
        out_shape=jax.ShapeDtypeStruct((SEQ, D_MODEL), jnp.float32),
        scratch_shapes=[pltpu.VMEM((SEQ, D_MODEL), jnp.float32)],
        interpret=interpret,
    )(x2d, W1, b1r, W2, b2r, wfull)
    return out


def kernel(x, w_gate, W1, b1, W2, b2, interpret=False):
    b, s, d = x.shape
    x2d = x.reshape(s, d)
    wg_p = jnp.concatenate(
        [w_gate.T, jnp.zeros((D_MODEL, EPAD - N_EXPERTS), jnp.float32)],
        axis=1)
    b1r = b1.reshape(N_EXPERTS, 1, FFN)
    b2r = b2.reshape(N_EXPERTS, 1, D_MODEL)
    out = _moe(x2d, wg_p, W1, b1r, W2, b2r, interpret=interpret)
    return out.reshape(b, s, d)


# speedup vs baseline: 2.6783x; 1.4875x over previous
import functools

import jax
import jax.numpy as jnp
from jax import lax
from jax.experimental import pallas as pl
from jax.experimental.pallas import tpu as pltpu
from jax.experimental.pallas import tpu_sc as plsc

D_MODEL = 1024
N_EXPERTS = 8
FFN = 4096
SEQ = 2048
EPAD = 128
FB = 512
NFB = FFN // FB
TILE = 256
NT = 24
PADTOT = NT * TILE

NC = 2
NS = 16
NW = NC * NS
CHUNK = 64


def _erf(x):
    a1, a2, a3, a4, a5 = (0.254829592, -0.284496736, 1.421413741,
                          -1.453152027, 1.061405429)
    p = 0.3275911
    ax = jnp.abs(x)
    t = 1.0 / (1.0 + p * ax)
    poly = ((((a5 * t + a4) * t + a3) * t + a2) * t + a1) * t
    y = 1.0 - poly * jnp.exp(-ax * ax)
    return jnp.sign(x) * y


def _gelu(x):
    return 0.5 * x * (1.0 + _erf(x * 0.7071067811865476))


def _gating_body(x_ref, wg_ref, pack_ref, te_ref):
    x = x_ref[...]
    logits = jnp.dot(x, wg_ref[...], preferred_element_type=jnp.float32)
    col = lax.broadcasted_iota(jnp.int32, (SEQ, EPAD), 1)
    valid = col < N_EXPERTS
    neg = jnp.float32(-1e30)
    l = jnp.where(valid, logits, neg)
    m1 = jnp.max(l, axis=1, keepdims=True)
    i1 = jnp.min(jnp.where(l == m1, col, EPAD), axis=1, keepdims=True)
    l2 = jnp.where(col == i1, neg, l)
    m2 = jnp.max(l2, axis=1, keepdims=True)
    i2 = jnp.min(jnp.where(l2 == m2, col, EPAD), axis=1, keepdims=True)
    s = jnp.where(valid, jnp.exp(l - m1), 0.0)
    den = jnp.sum(s, axis=1, keepdims=True)
    p1 = 1.0 / den
    p2 = jnp.exp(m2 - m1) / den

    o0 = (col == i1).astype(jnp.float32)
    o1 = (col == i2).astype(jnp.float32)
    r_io = lax.broadcasted_iota(jnp.int32, (SEQ, SEQ), 0)
    c_io = lax.broadcasted_iota(jnp.int32, (SEQ, SEQ), 1)
    ltri = (r_io >= c_io).astype(jnp.float32)
    c0 = jnp.dot(ltri, o0, preferred_element_type=jnp.float32)
    c1 = jnp.dot(ltri, o1, preferred_element_type=jnp.float32)
    c0last = c0[SEQ - 1:SEQ, :]
    c1last = c1[SEQ - 1:SEQ, :]
    cnt = (c0last + c1last).astype(jnp.int32)
    pc = ((cnt + (TILE - 1)) // TILE) * TILE
    pc_f = pc.astype(jnp.float32)
    r8 = lax.broadcasted_iota(jnp.int32, (EPAD, EPAD), 0)
    c8 = lax.broadcasted_iota(jnp.int32, (EPAD, EPAD), 1)
    ustri = (r8 < c8).astype(jnp.float32)
    poff = jnp.dot(pc_f, ustri, preferred_element_type=jnp.float32)

    pos0 = jnp.sum((poff + c0) * o0, axis=1, keepdims=True) - 1.0
    pos1 = jnp.sum((poff + c0last + c1) * o1, axis=1, keepdims=True) - 1.0

    pack_ref[...] = jnp.where(
        col == 0, pos0,
        jnp.where(col == 1, pos1,
                  jnp.where(col == 2, p1, jnp.where(col == 3, p2, 0.0))))

    pend = poff + pc_f
    tio = lax.broadcasted_iota(jnp.int32, (32, EPAD), 0)
    col32 = lax.broadcasted_iota(jnp.int32, (32, EPAD), 1)
    m = ((tio * TILE).astype(jnp.float32) >= pend) & (col32 < N_EXPERTS)
    te = jnp.sum(m.astype(jnp.int32), axis=1, keepdims=True)
    te_ref[...] = jnp.broadcast_to(te, (32, EPAD))


def _dispatch_body(x_hbm, posm_hbm, xs_hbm, idx_v, rows_v, sem):
    wid = lax.axis_index("s") * NC + lax.axis_index("c")
    slot = wid // NS
    t0 = (wid % NS) * (SEQ // NS)
    for chunk in range(SEQ // NS // CHUNK):
        base = t0 + chunk * CHUNK
        pltpu.sync_copy(posm_hbm.at[slot, pl.ds(base, CHUNK)], idx_v)
        pltpu.sync_copy(x_hbm.at[pl.ds(base, CHUNK)], rows_v)
        pltpu.async_copy(rows_v, xs_hbm.at[idx_v], sem).wait()


def _dispatch(x2d, posm):
    mesh = plsc.VectorSubcoreMesh(core_axis_name="c", subcore_axis_name="s")
    f = pl.kernel(
        _dispatch_body,
        mesh=mesh,
        out_type=jax.ShapeDtypeStruct((PADTOT, D_MODEL), jnp.float32),
        scratch_types=[
            pltpu.VMEM((CHUNK,), jnp.int32),
            pltpu.VMEM((CHUNK, D_MODEL), jnp.float32),
            pltpu.SemaphoreType.DMA,
        ],
    )
    return f(x2d, posm)


def _ffn_body(te_ref, xs_ref, w1_ref, b1_ref, w2_ref, b2_ref, out_ref):
    t = pl.program_id(0)
    f = pl.program_id(1)
    tile_ok = te_ref[t] < N_EXPERTS

    @pl.when(tile_ok)
    def _():
        h = jnp.dot(xs_ref[0], w1_ref[0], preferred_element_type=jnp.float32)
        h = _gelu(h + b1_ref[0])
        part = jnp.dot(h, w2_ref[0], preferred_element_type=jnp.float32)

        @pl.when(f == 0)
        def _():
            out_ref[0] = part

        @pl.when(f > 0)
        def _():
            out_ref[0] += part

        @pl.when(f == NFB - 1)
        def _():
            out_ref[0] += b2_ref[0]


def _ffn(te, xs3, W1, b1r, W2, b2r):
    def e_of(te_ref, t):
        return jnp.minimum(te_ref[t], N_EXPERTS - 1)

    def ok(te_ref, t):
        return te_ref[t] < N_EXPERTS

    grid_spec = pltpu.PrefetchScalarGridSpec(
        num_scalar_prefetch=1,
        grid=(NT, NFB),
        in_specs=[
            pl.BlockSpec((1, TILE, D_MODEL),
                         lambda t, f, te: (jnp.where(ok(te, t), t, 0), 0, 0)),
            pl.BlockSpec((1, D_MODEL, FB),
                         lambda t, f, te: (e_of(te, t), 0,
                                           jnp.where(ok(te, t), f, 0))),
            pl.BlockSpec((1, 1, FB),
                         lambda t, f, te: (e_of(te, t), 0,
                                           jnp.where(ok(te, t), f, 0))),
            pl.BlockSpec((1, FB, D_MODEL),
                         lambda t, f, te: (e_of(te, t),
                                           jnp.where(ok(te, t), f, 0), 0)),
            pl.BlockSpec((1, 1, D_MODEL),
                         lambda t, f, te: (e_of(te, t), 0, 0)),
        ],
        out_specs=pl.BlockSpec(
            (1, TILE, D_MODEL),
            lambda t, f, te: (jnp.where(ok(te, t), t, NT), 0, 0)),
    )
    return pl.pallas_call(
        _ffn_body,
        grid_spec=grid_spec,
        out_shape=jax.ShapeDtypeStruct((NT + 1, TILE, D_MODEL), jnp.float32),
    )(te, xs3, W1, b1r, W2, b2r)


def _gather_body(ys_hbm, posm_hbm, g_hbm, idx_v, rows_v, sem):
    wid = lax.axis_index("s") * NC + lax.axis_index("c")
    slot = wid // NS
    t0 = (wid % NS) * (SEQ // NS)
    for chunk in range(SEQ // NS // CHUNK):
        base = t0 + chunk * CHUNK
        pltpu.sync_copy(posm_hbm.at[slot, pl.ds(base, CHUNK)], idx_v)
        pltpu.async_copy(ys_hbm.at[idx_v], rows_v, sem).wait()
        pltpu.sync_copy(rows_v, g_hbm.at[slot, pl.ds(base, CHUNK)])


def _gather2(ysf, posm):
    mesh = plsc.VectorSubcoreMesh(core_axis_name="c", subcore_axis_name="s")
    f = pl.kernel(
        _gather_body,
        mesh=mesh,
        out_type=jax.ShapeDtypeStruct((2, SEQ, D_MODEL), jnp.float32),
        scratch_types=[
            pltpu.VMEM((CHUNK,), jnp.int32),
            pltpu.VMEM((CHUNK, D_MODEL), jnp.float32),
            pltpu.SemaphoreType.DMA,
        ],
    )
    return f(ysf, posm)


def _combine_body(g_ref, pack_ref, out_ref):
    w0 = pack_ref[:, 2:3]
    w1 = pack_ref[:, 3:4]
    out_ref[...] = w0 * g_ref[0] + w1 * g_ref[1]


def _combine(g, pack):
    return pl.pallas_call(
        _combine_body,
        out_shape=jax.ShapeDtypeStruct((SEQ, D_MODEL), jnp.float32),
    )(g, pack)


def kernel(x, w_gate, W1, b1, W2, b2):
    b, s, d = x.shape
    x2d = x.reshape(s, d)
    wg_p = jnp.concatenate(
        [w_gate.T, jnp.zeros((D_MODEL, EPAD - N_EXPERTS), jnp.float32)],
        axis=1)
    b1r = b1.reshape(N_EXPERTS, 1, FFN)
    b2r = b2.reshape(N_EXPERTS, 1, D_MODEL)

    pack, te_out = pl.pallas_call(
        _gating_body,
        out_shape=[
            jax.ShapeDtypeStruct((SEQ, EPAD), jnp.float32),
            jax.ShapeDtypeStruct((32, EPAD), jnp.int32),
        ],
    )(x2d, wg_p)

    posm = jnp.stack([pack[:, 0], pack[:, 1]]).astype(jnp.int32)
    te = te_out[:NT, 0]

    xs = _dispatch(x2d, posm)
    xs3 = xs.reshape(NT, TILE, D_MODEL)

    ys = _ffn(te, xs3, W1, b1r, W2, b2r)
    ysf = ys.reshape((NT + 1) * TILE, D_MODEL)

    g = _gather2(ysf, posm)
    out = _combine(g, pack)
    return out.reshape(b, s, d)


# speedup vs baseline: 3.0999x; 1.1574x over previous
import functools

import jax
import jax.numpy as jnp
from jax import lax
from jax.experimental import pallas as pl
from jax.experimental.pallas import tpu as pltpu
from jax.experimental.pallas import tpu_sc as plsc

D_MODEL = 1024
N_EXPERTS = 8
FFN = 4096
SEQ = 2048
EPAD = 128
FB = 1024
NFB = FFN // FB
TILE = 256
NT = 24
PADTOT = NT * TILE

NC = 2
NS = 16
NW = NC * NS
CHUNK = 64


def _erf(x):
    a1, a2, a3, a4, a5 = (0.254829592, -0.284496736, 1.421413741,
                          -1.453152027, 1.061405429)
    p = 0.3275911
    ax = jnp.abs(x)
    t = 1.0 / (1.0 + p * ax)
    poly = ((((a5 * t + a4) * t + a3) * t + a2) * t + a1) * t
    y = 1.0 - poly * jnp.exp(-ax * ax)
    return jnp.sign(x) * y


def _gelu(x):
    return 0.5 * x * (1.0 + _erf(x * 0.7071067811865476))


def _gating_body(x_ref, wg_ref, pack_ref, te_ref):
    x = x_ref[...]
    logits = jnp.dot(x, wg_ref[...], preferred_element_type=jnp.float32)
    col = lax.broadcasted_iota(jnp.int32, (SEQ, EPAD), 1)
    valid = col < N_EXPERTS
    neg = jnp.float32(-1e30)
    l = jnp.where(valid, logits, neg)
    m1 = jnp.max(l, axis=1, keepdims=True)
    i1 = jnp.min(jnp.where(l == m1, col, EPAD), axis=1, keepdims=True)
    l2 = jnp.where(col == i1, neg, l)
    m2 = jnp.max(l2, axis=1, keepdims=True)
    i2 = jnp.min(jnp.where(l2 == m2, col, EPAD), axis=1, keepdims=True)
    s = jnp.where(valid, jnp.exp(l - m1), 0.0)
    den = jnp.sum(s, axis=1, keepdims=True)
    p1 = 1.0 / den
    p2 = jnp.exp(m2 - m1) / den

    o0 = (col == i1).astype(jnp.float32)
    o1 = (col == i2).astype(jnp.float32)
    r_io = lax.broadcasted_iota(jnp.int32, (SEQ, SEQ), 0)
    c_io = lax.broadcasted_iota(jnp.int32, (SEQ, SEQ), 1)
    ltri = (r_io >= c_io).astype(jnp.float32)
    c0 = jnp.dot(ltri, o0, preferred_element_type=jnp.float32)
    c1 = jnp.dot(ltri, o1, preferred_element_type=jnp.float32)
    c0last = c0[SEQ - 1:SEQ, :]
    c1last = c1[SEQ - 1:SEQ, :]
    cnt = (c0last + c1last).astype(jnp.int32)
    pc = ((cnt + (TILE - 1)) // TILE) * TILE
    pc_f = pc.astype(jnp.float32)
    r8 = lax.broadcasted_iota(jnp.int32, (EPAD, EPAD), 0)
    c8 = lax.broadcasted_iota(jnp.int32, (EPAD, EPAD), 1)
    ustri = (r8 < c8).astype(jnp.float32)
    poff = jnp.dot(pc_f, ustri, preferred_element_type=jnp.float32)

    pos0 = jnp.sum((poff + c0) * o0, axis=1, keepdims=True) - 1.0
    pos1 = jnp.sum((poff + c0last + c1) * o1, axis=1, keepdims=True) - 1.0

    pack_ref[...] = jnp.where(
        col == 0, pos0,
        jnp.where(col == 1, pos1,
                  jnp.where(col == 2, p1, jnp.where(col == 3, p2, 0.0))))

    pend = poff + pc_f
    tio = lax.broadcasted_iota(jnp.int32, (32, EPAD), 0)
    col32 = lax.broadcasted_iota(jnp.int32, (32, EPAD), 1)
    m = ((tio * TILE).astype(jnp.float32) >= pend) & (col32 < N_EXPERTS)
    te = jnp.sum(m.astype(jnp.int32), axis=1, keepdims=True)
    te_ref[...] = jnp.broadcast_to(te, (32, EPAD))


def _dispatch_body(x_hbm, posm_hbm, xs_hbm, idx_v, rows_v, sem):
    wid = lax.axis_index("s") * NC + lax.axis_index("c")
    slot = wid // NS
    t0 = (wid % NS) * (SEQ // NS)
    for chunk in range(SEQ // NS // CHUNK):
        base = t0 + chunk * CHUNK
        pltpu.sync_copy(posm_hbm.at[slot, pl.ds(base, CHUNK)], idx_v)
        pltpu.sync_copy(x_hbm.at[pl.ds(base, CHUNK)], rows_v)
        pltpu.async_copy(rows_v, xs_hbm.at[idx_v], sem).wait()


def _dispatch(x2d, posm):
    mesh = plsc.VectorSubcoreMesh(core_axis_name="c", subcore_axis_name="s")
    f = pl.kernel(
        _dispatch_body,
        mesh=mesh,
        out_type=jax.ShapeDtypeStruct((PADTOT, D_MODEL), jnp.float32),
        scratch_types=[
            pltpu.VMEM((CHUNK,), jnp.int32),
            pltpu.VMEM((CHUNK, D_MODEL), jnp.float32),
            pltpu.SemaphoreType.DMA,
        ],
    )
    return f(x2d, posm)


def _ffn_body(te_ref, xs_ref, w1_ref, b1_ref, w2_ref, b2_ref, out_ref,
              acc_ref):
    f = pl.program_id(0)
    t = pl.program_id(1)
    tile_ok = te_ref[t] < N_EXPERTS

    @pl.when(tile_ok)
    def _():
        h = jnp.dot(xs_ref[0], w1_ref[0], preferred_element_type=jnp.float32)
        h = _gelu(h + b1_ref[0])
        part = jnp.dot(h, w2_ref[0], preferred_element_type=jnp.float32)

        @pl.when(f == 0)
        def _():
            acc_ref[t] = part

        @pl.when(jnp.logical_and(f > 0, f < NFB - 1))
        def _():
            acc_ref[t] += part

        @pl.when(f == NFB - 1)
        def _():
            out_ref[0] = acc_ref[t] + part + b2_ref[0]


def _ffn(te, xs3, W1, b1r, W2, b2r):
    def e_of(te_ref, t):
        return jnp.minimum(te_ref[t], N_EXPERTS - 1)

    def ok(te_ref, t):
        return te_ref[t] < N_EXPERTS

    grid_spec = pltpu.PrefetchScalarGridSpec(
        num_scalar_prefetch=1,
        grid=(NFB, NT),
        in_specs=[
            pl.BlockSpec((1, TILE, D_MODEL),
                         lambda f, t, te: (jnp.where(ok(te, t), t, 0), 0, 0)),
            pl.BlockSpec((1, D_MODEL, FB),
                         lambda f, t, te: (e_of(te, t), 0,
                                           jnp.where(ok(te, t), f, 0))),
            pl.BlockSpec((1, 1, FB),
                         lambda f, t, te: (e_of(te, t), 0,
                                           jnp.where(ok(te, t), f, 0))),
            pl.BlockSpec((1, FB, D_MODEL),
                         lambda f, t, te: (e_of(te, t),
                                           jnp.where(ok(te, t), f, 0), 0)),
            pl.BlockSpec((1, 1, D_MODEL),
                         lambda f, t, te: (e_of(te, t), 0, 0)),
        ],
        out_specs=pl.BlockSpec(
            (1, TILE, D_MODEL),
            lambda f, t, te: (jnp.where(
                jnp.logical_and(ok(te, t), f == NFB - 1), t, NT), 0, 0)),
        scratch_shapes=[pltpu.VMEM((NT, TILE, D_MODEL), jnp.float32)],
    )
    return pl.pallas_call(
        _ffn_body,
        grid_spec=grid_spec,
        out_shape=jax.ShapeDtypeStruct((NT + 1, TILE, D_MODEL), jnp.float32),
    )(te, xs3, W1, b1r, W2, b2r)


def _gather_body(ys_hbm, posm_hbm, g_hbm, idx_v, rows_v, sem):
    wid = lax.axis_index("s") * NC + lax.axis_index("c")
    slot = wid // NS
    t0 = (wid % NS) * (SEQ // NS)
    for chunk in range(SEQ // NS // CHUNK):
        base = t0 + chunk * CHUNK
        pltpu.sync_copy(posm_hbm.at[slot, pl.ds(base, CHUNK)], idx_v)
        pltpu.async_copy(ys_hbm.at[idx_v], rows_v, sem).wait()
        pltpu.sync_copy(rows_v, g_hbm.at[slot, pl.ds(base, CHUNK)])


def _gather2(ysf, posm):
    mesh = plsc.VectorSubcoreMesh(core_axis_name="c", subcore_axis_name="s")
    f = pl.kernel(
        _gather_body,
        mesh=mesh,
        out_type=jax.ShapeDtypeStruct((2, SEQ, D_MODEL), jnp.float32),
        scratch_types=[
            pltpu.VMEM((CHUNK,), jnp.int32),
            pltpu.VMEM((CHUNK, D_MODEL), jnp.float32),
            pltpu.SemaphoreType.DMA,
        ],
    )
    return f(ysf, posm)


def _combine_body(g_ref, pack_ref, out_ref):
    w0 = pack_ref[:, 2:3]
    w1 = pack_ref[:, 3:4]
    out_ref[...] = w0 * g_ref[0] + w1 * g_ref[1]


def _combine(g, pack):
    return pl.pallas_call(
        _combine_body,
        out_shape=jax.ShapeDtypeStruct((SEQ, D_MODEL), jnp.float32),
    )(g, pack)


def kernel(x, w_gate, W1, b1, W2, b2):
    b, s, d = x.shape
    x2d = x.reshape(s, d)
    wg_p = jnp.concatenate(
        [w_gate.T, jnp.zeros((D_MODEL, EPAD - N_EXPERTS), jnp.float32)],
        axis=1)
    b1r = b1.reshape(N_EXPERTS, 1, FFN)
    b2r = b2.reshape(N_EXPERTS, 1, D_MODEL)

    pack, te_out = pl.pallas_call(
        _gating_body,
        out_shape=[
            jax.ShapeDtypeStruct((SEQ, EPAD), jnp.float32),
            jax.ShapeDtypeStruct((32, EPAD), jnp.int32),
        ],
    )(x2d, wg_p)

    posm = jnp.stack([pack[:, 0], pack[:, 1]]).astype(jnp.int32)
    te = te_out[:NT, 0]

    xs = _dispatch(x2d, posm)
    xs3 = xs.reshape(NT, TILE, D_MODEL)

    ys = _ffn(te, xs3, W1, b1r, W2, b2r)
    ysf = ys.reshape((NT + 1) * TILE, D_MODEL)

    g = _gather2(ysf, posm)
    out = _combine(g, pack)
    return out.reshape(b, s, d)


# speedup vs baseline: 3.5856x; 1.1567x over previous
import functools

import jax
import jax.numpy as jnp
from jax import lax
from jax.experimental import pallas as pl
from jax.experimental.pallas import tpu as pltpu
from jax.experimental.pallas import tpu_sc as plsc

D_MODEL = 1024
N_EXPERTS = 8
FFN = 4096
SEQ = 2048
EPAD = 128
FB = 1024
NFB = FFN // FB
TILE = 256
NT = 24
PADTOT = NT * TILE

NC = 2
NS = 16
NW = NC * NS
CHUNK = 64


def _erf(x):
    a1, a2, a3, a4, a5 = (0.254829592, -0.284496736, 1.421413741,
                          -1.453152027, 1.061405429)
    p = 0.3275911
    ax = jnp.abs(x)
    t = 1.0 / (1.0 + p * ax)
    poly = ((((a5 * t + a4) * t + a3) * t + a2) * t + a1) * t
    y = 1.0 - poly * jnp.exp(-ax * ax)
    return jnp.sign(x) * y


def _gelu(x):
    return 0.5 * x * (1.0 + lax.erf(x * 0.7071067811865476))


def _gating_body(x_ref, wg_ref, pack_ref, te_ref, posr_ref):
    x = x_ref[...]
    logits8 = lax.dot_general(x, wg_ref[...], (((1,), (1,)), ((), ())),
                              preferred_element_type=jnp.float32)
    col = lax.broadcasted_iota(jnp.int32, (SEQ, EPAD), 1)
    valid = col < N_EXPERTS
    neg = jnp.float32(-1e30)
    l = jnp.concatenate(
        [logits8, jnp.full((SEQ, EPAD - N_EXPERTS), neg, jnp.float32)],
        axis=1)
    m1 = jnp.max(l, axis=1, keepdims=True)
    i1 = jnp.min(jnp.where(l == m1, col, EPAD), axis=1, keepdims=True)
    l2 = jnp.where(col == i1, neg, l)
    m2 = jnp.max(l2, axis=1, keepdims=True)
    i2 = jnp.min(jnp.where(l2 == m2, col, EPAD), axis=1, keepdims=True)
    s = jnp.where(valid, jnp.exp(l - m1), 0.0)
    den = jnp.sum(s, axis=1, keepdims=True)
    p1 = 1.0 / den
    p2 = jnp.exp(m2 - m1) / den

    o0 = (col == i1).astype(jnp.float32)
    o1 = (col == i2).astype(jnp.float32)
    r_io = lax.broadcasted_iota(jnp.int32, (SEQ, SEQ), 0)
    c_io = lax.broadcasted_iota(jnp.int32, (SEQ, SEQ), 1)
    ltri = (r_io >= c_io).astype(jnp.float32)
    c0 = jnp.dot(ltri, o0, preferred_element_type=jnp.float32)
    c1 = jnp.dot(ltri, o1, preferred_element_type=jnp.float32)
    c0last = c0[SEQ - 1:SEQ, :]
    c1last = c1[SEQ - 1:SEQ, :]
    cnt = (c0last + c1last).astype(jnp.int32)
    pc = ((cnt + (TILE - 1)) // TILE) * TILE
    pc_f = pc.astype(jnp.float32)
    r8 = lax.broadcasted_iota(jnp.int32, (EPAD, EPAD), 0)
    c8 = lax.broadcasted_iota(jnp.int32, (EPAD, EPAD), 1)
    ustri = (r8 < c8).astype(jnp.float32)
    poff = jnp.dot(pc_f, ustri, preferred_element_type=jnp.float32)

    pos0 = jnp.sum((poff + c0) * o0, axis=1, keepdims=True) - 1.0
    pos1 = jnp.sum((poff + c0last + c1) * o1, axis=1, keepdims=True) - 1.0

    pack_ref[...] = jnp.where(
        col == 0, pos0,
        jnp.where(col == 1, pos1,
                  jnp.where(col == 2, p1, jnp.where(col == 3, p2, 0.0))))

    pend = poff + pc_f
    tio = lax.broadcasted_iota(jnp.int32, (32, EPAD), 0)
    col32 = lax.broadcasted_iota(jnp.int32, (32, EPAD), 1)
    m = ((tio * TILE).astype(jnp.float32) >= pend) & (col32 < N_EXPERTS)
    te = jnp.sum(m.astype(jnp.int32), axis=1, keepdims=True)
    te_ref[...] = jnp.broadcast_to(te, (32, EPAD))

    p8 = jnp.concatenate(
        [pos0, pos1, jnp.zeros((SEQ, 6), jnp.float32)], axis=1)
    posr_ref[...] = jnp.transpose(p8, (1, 0)).astype(jnp.int32)


def _dispatch_body(x_hbm, posm_hbm, xs_hbm, idx_v, rows_v, sem):
    wid = lax.axis_index("s") * NC + lax.axis_index("c")
    slot = wid // NS
    t0 = (wid % NS) * (SEQ // NS)
    for chunk in range(SEQ // NS // CHUNK):
        base = t0 + chunk * CHUNK
        pltpu.sync_copy(posm_hbm.at[slot, pl.ds(base, CHUNK)], idx_v)
        pltpu.sync_copy(x_hbm.at[pl.ds(base, CHUNK)], rows_v)
        pltpu.async_copy(rows_v, xs_hbm.at[idx_v], sem).wait()


def _dispatch(x2d, posm):
    mesh = plsc.VectorSubcoreMesh(core_axis_name="c", subcore_axis_name="s")
    f = pl.kernel(
        _dispatch_body,
        mesh=mesh,
        out_type=jax.ShapeDtypeStruct((PADTOT, D_MODEL), jnp.float32),
        scratch_types=[
            pltpu.VMEM((CHUNK,), jnp.int32),
            pltpu.VMEM((CHUNK, D_MODEL), jnp.float32),
            pltpu.SemaphoreType.DMA,
        ],
    )
    return f(x2d, posm)


def _ffn_body(te_ref, xs_ref, w1_ref, b1_ref, w2_ref, b2_ref, out_ref,
              acc_ref):
    f = pl.program_id(0)
    t = pl.program_id(1)
    tile_ok = te_ref[t] < N_EXPERTS

    @pl.when(tile_ok)
    def _():
        h = jnp.dot(xs_ref[0], w1_ref[0], preferred_element_type=jnp.float32)
        h = _gelu(h + b1_ref[0])
        part = jnp.dot(h, w2_ref[0], preferred_element_type=jnp.float32)

        @pl.when(f == 0)
        def _():
            acc_ref[t] = part

        @pl.when(jnp.logical_and(f > 0, f < NFB - 1))
        def _():
            acc_ref[t] += part

        @pl.when(f == NFB - 1)
        def _():
            out_ref[0] = acc_ref[t] + part + b2_ref[0]


def _ffn(te, xs3, W1, b1r, W2, b2r):
    def e_of(te_ref, t):
        return jnp.minimum(te_ref[t], N_EXPERTS - 1)

    def ok(te_ref, t):
        return te_ref[t] < N_EXPERTS

    grid_spec = pltpu.PrefetchScalarGridSpec(
        num_scalar_prefetch=1,
        grid=(NFB, NT),
        in_specs=[
            pl.BlockSpec((1, TILE, D_MODEL),
                         lambda f, t, te: (jnp.where(ok(te, t), t, 0), 0, 0)),
            pl.BlockSpec((1, D_MODEL, FB),
                         lambda f, t, te: (e_of(te, t), 0,
                                           jnp.where(ok(te, t), f, 0))),
            pl.BlockSpec((1, 1, FB),
                         lambda f, t, te: (e_of(te, t), 0,
                                           jnp.where(ok(te, t), f, 0))),
            pl.BlockSpec((1, FB, D_MODEL),
                         lambda f, t, te: (e_of(te, t),
                                           jnp.where(ok(te, t), f, 0), 0)),
            pl.BlockSpec((1, 1, D_MODEL),
                         lambda f, t, te: (e_of(te, t), 0, 0)),
        ],
        out_specs=pl.BlockSpec(
            (1, TILE, D_MODEL),
            lambda f, t, te: (jnp.where(
                jnp.logical_and(ok(te, t), f == NFB - 1), t, NT), 0, 0)),
        scratch_shapes=[pltpu.VMEM((NT, TILE, D_MODEL), jnp.float32)],
    )
    return pl.pallas_call(
        _ffn_body,
        grid_spec=grid_spec,
        out_shape=jax.ShapeDtypeStruct((NT + 1, TILE, D_MODEL), jnp.float32),
    )(te, xs3, W1, b1r, W2, b2r)


def _gather_body(ys_hbm, posm_hbm, g_hbm, idx_v, rows_v, sem):
    wid = lax.axis_index("s") * NC + lax.axis_index("c")
    slot = wid // NS
    t0 = (wid % NS) * (SEQ // NS)
    for chunk in range(SEQ // NS // CHUNK):
        base = t0 + chunk * CHUNK
        pltpu.sync_copy(posm_hbm.at[slot, pl.ds(base, CHUNK)], idx_v)
        pltpu.async_copy(ys_hbm.at[idx_v], rows_v, sem).wait()
        pltpu.sync_copy(rows_v, g_hbm.at[slot, pl.ds(base, CHUNK)])


def _gather2(ysf, posm):
    mesh = plsc.VectorSubcoreMesh(core_axis_name="c", subcore_axis_name="s")
    f = pl.kernel(
        _gather_body,
        mesh=mesh,
        out_type=jax.ShapeDtypeStruct((2, SEQ, D_MODEL), jnp.float32),
        scratch_types=[
            pltpu.VMEM((CHUNK,), jnp.int32),
            pltpu.VMEM((CHUNK, D_MODEL), jnp.float32),
            pltpu.SemaphoreType.DMA,
        ],
    )
    return f(ysf, posm)


def _combine_body(g_ref, pack_ref, out_ref):
    w0 = pack_ref[:, 2:3]
    w1 = pack_ref[:, 3:4]
    out_ref[...] = w0 * g_ref[0] + w1 * g_ref[1]


def _combine(g, pack):
    return pl.pallas_call(
        _combine_body,
        out_shape=jax.ShapeDtypeStruct((SEQ, D_MODEL), jnp.float32),
    )(g, pack)


def kernel(x, w_gate, W1, b1, W2, b2):
    b, s, d = x.shape
    x2d = x.reshape(s, d)
    b1r = b1.reshape(N_EXPERTS, 1, FFN)
    b2r = b2.reshape(N_EXPERTS, 1, D_MODEL)

    pack, te_out, posr = pl.pallas_call(
        _gating_body,
        out_shape=[
            jax.ShapeDtypeStruct((SEQ, EPAD), jnp.float32),
            jax.ShapeDtypeStruct((32, EPAD), jnp.int32),
            jax.ShapeDtypeStruct((8, SEQ), jnp.int32),
        ],
    )(x2d, w_gate)

    te = te_out[:NT, 0]

    xs = _dispatch(x2d, posr)
    xs3 = xs.reshape(NT, TILE, D_MODEL)

    ys = _ffn(te, xs3, W1, b1r, W2, b2r)
    ysf = ys.reshape((NT + 1) * TILE, D_MODEL)

    g = _gather2(ysf, posr)
    out = _combine(g, pack)
    return out.reshape(b, s, d)
